# baseline (speedup 1.0000x reference)
import functools

import jax
import jax.numpy as jnp
from jax import lax
from jax.experimental import pallas as pl
from jax.experimental.pallas import tpu as pltpu
from jax.experimental.pallas import tpu_sc as plsc

NUM_VOCAB = 1000000
EMBED = 64
HIDDEN = 128
NUM_CLASSES = 2
BATCH = 4096
SEQ = 200

NUM_CORES = 2
NUM_SUBCORES = 16
NUM_WORKERS = NUM_CORES * NUM_SUBCORES
BPW = BATCH // NUM_WORKERS
S0 = 128
S1 = SEQ - S0
LANES = 16
NCH = EMBED // LANES


def _sc_pool_kernel(table_hbm, idx_hbm, out_hbm, idx_v, rows_v, acc_v,
                    sem0, sem1):
    wid = lax.axis_index("s") * NUM_CORES + lax.axis_index("c")
    base = wid * (BPW * SEQ)
    pltpu.sync_copy(idx_hbm.at[pl.ds(base, BPW * SEQ)], idx_v)

    sems = (sem0, sem1)

    def start(b, buf):
        off = b * SEQ
        pltpu.async_copy(table_hbm.at[idx_v.at[pl.ds(off, S0)]],
                         rows_v.at[buf, pl.ds(0, S0), :], sems[buf])
        pltpu.async_copy(table_hbm.at[idx_v.at[pl.ds(off + S0, S1)]],
                         rows_v.at[buf, pl.ds(S0, S1), :], sems[buf])

    def wait(buf):
        pltpu.make_async_copy(table_hbm.at[pl.ds(0, SEQ), :],
                              rows_v.at[buf], sems[buf]).wait()

    start(0, 0)

    @pl.loop(0, BPW, step=2)
    def _outer(g):
        for k in range(2):
            b = g + k
            buf = k

            @pl.when(b + 1 < BPW)
            def _():
                start(b + 1, buf ^ 1)

            wait(buf)

            def body(j, accs):
                r = rows_v
                return tuple(
                    accs[c] + r[buf, j, pl.ds(c * LANES, LANES)]
                    for c in range(NCH))

            zero = jnp.zeros((LANES,), jnp.float32)
            accs = lax.fori_loop(0, SEQ, body, (zero,) * NCH, unroll=4)
            for c in range(NCH):
                acc_v[b, pl.ds(c * LANES, LANES)] = accs[c] * (1.0 / SEQ)

    pltpu.sync_copy(acc_v, out_hbm.at[pl.ds(wid * BPW, BPW), :])


@jax.jit
def _sc_pool(table, idx_flat):
    mesh = plsc.VectorSubcoreMesh(core_axis_name="c", subcore_axis_name="s")
    f = functools.partial(
        pl.kernel,
        out_type=jax.ShapeDtypeStruct((BATCH, EMBED), jnp.float32),
        mesh=mesh,
        scratch_types=[
            pltpu.VMEM((BPW * SEQ,), jnp.int32),
            pltpu.VMEM((2, SEQ, EMBED), jnp.float32),
            pltpu.VMEM((BPW, EMBED), jnp.float32),
            pltpu.SemaphoreType.DMA,
            pltpu.SemaphoreType.DMA,
        ],
        compiler_params=pltpu.CompilerParams(use_tc_tiling_on_sc=False),
    )(_sc_pool_kernel)
    return f(table, idx_flat)


def _mlp_body(p_ref, w1_ref, b1_ref, g_ref, be_ref, w2_ref, b2_ref, o_ref):
    h = jnp.dot(p_ref[...], w1_ref[...],
                preferred_element_type=jnp.float32) + b1_ref[...]
    mu = jnp.mean(h, axis=0, keepdims=True)
    d = h - mu
    var = jnp.mean(d * d, axis=0, keepdims=True)
    hn = d * lax.rsqrt(var + 1e-5) * g_ref[...] + be_ref[...]
    hn = jnp.maximum(hn, 0.0)
    o_ref[...] = jnp.dot(hn, w2_ref[...],
                         preferred_element_type=jnp.float32) + b2_ref[...]


@jax.jit
def _mlp(pooled, W1, b1, gamma, beta, W2p, b2p):
    return pl.pallas_call(
        _mlp_body,
        out_shape=jax.ShapeDtypeStruct((BATCH, HIDDEN), jnp.float32),
    )(pooled, W1, b1, gamma, beta, W2p, b2p)


def kernel(x, table, W1, b1, gamma, beta, W2, b2):
    idx_flat = x.reshape(-1)
    pooled = _sc_pool(table, idx_flat)
    W2p = jnp.zeros((HIDDEN, HIDDEN), jnp.float32).at[:, :NUM_CLASSES].set(W2)
    b2p = jnp.zeros((1, HIDDEN), jnp.float32).at[:, :NUM_CLASSES].set(b2)
    out = _mlp(pooled, W1, b1.reshape(1, HIDDEN), gamma.reshape(1, HIDDEN),
               beta.reshape(1, HIDDEN), W2p, b2p)
    return out[:, :NUM_CLASSES]


# speedup vs baseline: 1.0280x; 1.0280x over previous
import functools

import jax
import jax.numpy as jnp
from jax import lax
from jax.experimental import pallas as pl
from jax.experimental.pallas import tpu as pltpu
from jax.experimental.pallas import tpu_sc as plsc

NUM_VOCAB = 1000000
EMBED = 64
HIDDEN = 128
NUM_CLASSES = 2
BATCH = 4096
SEQ = 200

NUM_CORES = 2
NUM_SUBCORES = 16
NUM_WORKERS = NUM_CORES * NUM_SUBCORES
BPW = BATCH // NUM_WORKERS
SG = 4
NGROUPS = SEQ // SG
LANES = 16
NCH = EMBED // LANES


def _sc_pool_kernel(table_hbm, idx_hbm, out_hbm, idx_v, rows_v, acc_v,
                    sem0, sem1):
    wid = lax.axis_index("s") * NUM_CORES + lax.axis_index("c")
    base = wid * BPW
    pltpu.sync_copy(idx_hbm.at[:, pl.ds(base, BPW)], idx_v)

    sems = (sem0, sem1)

    def copies(g, buf):
        return [
            pltpu.make_async_copy(table_hbm.at[idx_v.at[g * SG + kk]],
                                  rows_v.at[buf, kk], sems[buf])
            for kk in range(SG)
        ]

    zero = jnp.zeros((LANES,), jnp.float32)

    @pl.loop(0, BPW)
    def _zero(b):
        for c in range(NCH):
            acc_v[b, pl.ds(c * LANES, LANES)] = zero

    for c in copies(0, 0):
        c.start()

    @pl.loop(0, NGROUPS, step=2)
    def _outer(g2):
        for k in range(2):
            g = g2 + k

            @pl.when(g + 1 < NGROUPS)
            def _():
                for c in copies(g + 1, k ^ 1):
                    c.start()

            for c in copies(g, k):
                c.wait()

            @pl.loop(0, BPW, unroll=2)
            def _acc(b):
                for c in range(NCH):
                    sl = pl.ds(c * LANES, LANES)
                    s01 = rows_v[k, 0, b, sl] + rows_v[k, 1, b, sl]
                    s23 = rows_v[k, 2, b, sl] + rows_v[k, 3, b, sl]
                    acc_v[b, sl] = acc_v[b, sl] + (s01 + s23)

    @pl.loop(0, BPW)
    def _scale(b):
        for c in range(NCH):
            sl = pl.ds(c * LANES, LANES)
            acc_v[b, sl] = acc_v[b, sl] * (1.0 / SEQ)

    pltpu.sync_copy(acc_v, out_hbm.at[pl.ds(base, BPW), :])


@jax.jit
def _sc_pool(table, idx_t):
    mesh = plsc.VectorSubcoreMesh(core_axis_name="c", subcore_axis_name="s")
    f = functools.partial(
        pl.kernel,
        out_type=jax.ShapeDtypeStruct((BATCH, EMBED), jnp.float32),
        mesh=mesh,
        scratch_types=[
            pltpu.VMEM((SEQ, BPW), jnp.int32),
            pltpu.VMEM((2, SG, BPW, EMBED), jnp.float32),
            pltpu.VMEM((BPW, EMBED), jnp.float32),
            pltpu.SemaphoreType.DMA,
            pltpu.SemaphoreType.DMA,
        ],
        compiler_params=pltpu.CompilerParams(use_tc_tiling_on_sc=False),
    )(_sc_pool_kernel)
    return f(table, idx_t)


def _mlp_body(p_ref, w1_ref, b1_ref, g_ref, be_ref, w2_ref, b2_ref, o_ref):
    h = jnp.dot(p_ref[...], w1_ref[...],
                preferred_element_type=jnp.float32) + b1_ref[...]
    mu = jnp.mean(h, axis=0, keepdims=True)
    d = h - mu
    var = jnp.mean(d * d, axis=0, keepdims=True)
    hn = d * lax.rsqrt(var + 1e-5) * g_ref[...] + be_ref[...]
    hn = jnp.maximum(hn, 0.0)
    o_ref[...] = jnp.dot(hn, w2_ref[...],
                         preferred_element_type=jnp.float32) + b2_ref[...]


@jax.jit
def _mlp(pooled, W1, b1, gamma, beta, W2p, b2p):
    return pl.pallas_call(
        _mlp_body,
        out_shape=jax.ShapeDtypeStruct((BATCH, HIDDEN), jnp.float32),
    )(pooled, W1, b1, gamma, beta, W2p, b2p)


def kernel(x, table, W1, b1, gamma, beta, W2, b2):
    idx_t = jnp.swapaxes(x[0], 0, 1)
    pooled = _sc_pool(table, idx_t)
    W2p = jnp.zeros((HIDDEN, HIDDEN), jnp.float32).at[:, :NUM_CLASSES].set(W2)
    b2p = jnp.zeros((1, HIDDEN), jnp.float32).at[:, :NUM_CLASSES].set(b2)
    out = _mlp(pooled, W1, b1.reshape(1, HIDDEN), gamma.reshape(1, HIDDEN),
               beta.reshape(1, HIDDEN), W2p, b2p)
    return out[:, :NUM_CLASSES]
